# Initial kernel scaffold; baseline (speedup 1.0000x reference)
import jax
import jax.numpy as jnp
from jax.experimental import pallas as pl


def kernel(x, edge_index, batch, embedding, gru_w_ih, gru_w_hh, W1, W2, b2, Wq, bq, Wt):
    raise NotImplementedError("write your pallas kernel here")


# speedup vs baseline: 5.3301x; 5.3301x over previous
import functools

import jax
import jax.numpy as jnp
from jax import lax
from jax.experimental import pallas as pl
from jax.experimental.pallas import tpu as pltpu
from jax.experimental.pallas import tpu_sc as plsc

G = 256

_NC = 2
_NS = 16
_CHUNK = 128


def _sc_edge_body(ncons, x_hbm, src_hbm, dst_hbm, emb_hbm, zeros_hbm, out_hbm,
                  msg_s, emb_s, x_s, src_v, dst_v, t_v, rows_v,
                  srcr_v, dstr_v, tr_v, rowsr_v):
    n_chunks, rem, per_worker, rows_per_tile = ncons
    c = lax.axis_index("c")
    s = lax.axis_index("s")

    @pl.when(s == 0)
    def _stage():
        pltpu.sync_copy(emb_hbm, emb_s)
        pltpu.sync_copy(x_hbm, x_s)

    pltpu.sync_copy(zeros_hbm, msg_s.at[pl.ds(s * rows_per_tile, rows_per_tile)])
    plsc.subcore_barrier()

    base0 = (c * _NS + s) * per_worker

    def chunk(j, carry):
        base = base0 + j * _CHUNK
        pltpu.sync_copy(src_hbm.at[pl.ds(base, _CHUNK)], src_v)
        pltpu.sync_copy(dst_hbm.at[pl.ds(base, _CHUNK)], dst_v)
        pltpu.sync_copy(x_s.at[src_v], t_v)
        pltpu.sync_copy(emb_s.at[t_v], rows_v)
        pltpu.sync_copy(rows_v, msg_s.at[dst_v], add=True)
        return carry

    lax.fori_loop(0, n_chunks, chunk, 0)

    if rem:
        base = base0 + n_chunks * _CHUNK
        pltpu.sync_copy(src_hbm.at[pl.ds(base, rem)], srcr_v)
        pltpu.sync_copy(dst_hbm.at[pl.ds(base, rem)], dstr_v)
        pltpu.sync_copy(x_s.at[srcr_v], tr_v)
        pltpu.sync_copy(emb_s.at[tr_v], rowsr_v)
        pltpu.sync_copy(rowsr_v, msg_s.at[dstr_v], add=True)

    plsc.subcore_barrier()
    pltpu.sync_copy(msg_s.at[pl.ds(s * rows_per_tile, rows_per_tile)],
                    out_hbm.at[c, pl.ds(s * rows_per_tile, rows_per_tile)])


def _sc_edge(x_flat, src, dst, embedding):
    N = x_flat.shape[0]
    E = src.shape[0]
    NT, H = embedding.shape
    per_worker = E // (_NC * _NS)
    assert per_worker * _NC * _NS == E
    n_chunks, rem = divmod(per_worker, _CHUNK)
    rows_per_tile = -(-N // (_NS * 8)) * 8
    n_pad = rows_per_tile * _NS
    rem_alloc = max(rem, 8)

    zeros = jnp.zeros((rows_per_tile, H), jnp.float32)
    mesh = plsc.VectorSubcoreMesh(core_axis_name="c", subcore_axis_name="s")
    fn = pl.kernel(
        functools.partial(_sc_edge_body, (n_chunks, rem, per_worker, rows_per_tile)),
        out_type=jax.ShapeDtypeStruct((_NC, n_pad, H), jnp.float32),
        mesh=mesh,
        scratch_types=[
            pltpu.VMEM_SHARED((n_pad, H), jnp.float32),
            pltpu.VMEM_SHARED((NT, H), jnp.float32),
            pltpu.VMEM_SHARED((N,), jnp.int32),
            pltpu.VMEM((_CHUNK,), jnp.int32),
            pltpu.VMEM((_CHUNK,), jnp.int32),
            pltpu.VMEM((_CHUNK,), jnp.int32),
            pltpu.VMEM((_CHUNK, H), jnp.float32),
            pltpu.VMEM((rem_alloc,), jnp.int32),
            pltpu.VMEM((rem_alloc,), jnp.int32),
            pltpu.VMEM((rem_alloc,), jnp.int32),
            pltpu.VMEM((rem_alloc, H), jnp.float32),
        ],
    )
    return fn(x_flat, src, dst, embedding, zeros)


def _dot_t(a, b):
    return lax.dot_general(a, b, (((1,), (1,)), ((), ())),
                           preferred_element_type=jnp.float32,
                           precision=lax.Precision.HIGHEST)


def _dot_n(a, b):
    return lax.dot_general(a, b, (((0,), (0,)), ((), ())),
                           preferred_element_type=jnp.float32,
                           precision=lax.Precision.HIGHEST)


def _dot(a, b):
    return lax.dot_general(a, b, (((1,), (0,)), ((), ())),
                           preferred_element_type=jnp.float32,
                           precision=lax.Precision.HIGHEST)


def _tc_body(msg_ref, xi_ref, bi_ref, emb_ref, wih_ref, whh_ref, w1_ref,
             w2_ref, b2_ref, wq_ref, bq_ref, wt_ref, out_ref,
             h_s, wl_s, wg_s, li_s):
    N, H = h_s.shape
    NT = emb_ref.shape[0]
    BLK = 1000
    NB = N // BLK

    li_s[...] = jnp.full((1, G), -1, jnp.int32)
    wl_s[...] = jnp.zeros((G, H), jnp.float32)
    wg_s[...] = jnp.zeros((G, H), jnp.float32)

    def phase_a(i, carry):
        ds = pl.ds(i * BLK, BLK)
        xb = xi_ref[ds, :]
        oh_t = (xb == lax.broadcasted_iota(jnp.int32, (BLK, NT), 1)
                ).astype(jnp.float32)
        emb_b = _dot(oh_t, emb_ref[...])
        msg_b = msg_ref[0, ds, :] + msg_ref[1, ds, :]
        gi = _dot_t(msg_b, wih_ref[...])
        gh = _dot_t(emb_b, whh_ref[...])
        r = jax.nn.sigmoid(gi[:, :H] + gh[:, :H])
        z = jax.nn.sigmoid(gi[:, H:2 * H] + gh[:, H:2 * H])
        n = jnp.tanh(gi[:, 2 * H:] + r * gh[:, 2 * H:])
        h_b = (1.0 - z) * n + z * emb_b
        h_s[ds, :] = h_b
        bb = bi_ref[ds, :]
        oh_g = bb == lax.broadcasted_iota(jnp.int32, (BLK, G), 1)
        nidx = lax.broadcasted_iota(jnp.int32, (BLK, 1), 0) + i * BLK
        li_s[...] = jnp.maximum(
            li_s[...], jnp.max(jnp.where(oh_g, nidx, -1), axis=0, keepdims=True))
        return carry

    lax.fori_loop(0, NB, phase_a, 0)
    li = jnp.maximum(li_s[...], 0)

    def phase_b(i, carry):
        ds = pl.ds(i * BLK, BLK)
        nidx = lax.broadcasted_iota(jnp.int32, (BLK, 1), 0) + i * BLK
        oh_l = (nidx == li).astype(jnp.float32)
        wl_s[...] += _dot_n(oh_l, h_s[ds, :])
        return carry

    lax.fori_loop(0, NB, phase_b, 0)

    def phase_c(i, carry):
        ds = pl.ds(i * BLK, BLK)
        bb = bi_ref[ds, :]
        oh_g = (bb == lax.broadcasted_iota(jnp.int32, (BLK, G), 1)
                ).astype(jnp.float32)
        h_b = h_s[ds, :]
        wgr = _dot(oh_g, wl_s[...])
        q1 = _dot_t(wgr, w1_ref[...])
        q2 = _dot_t(h_b, w2_ref[...]) + b2_ref[...]
        alpha = _dot_t(jax.nn.sigmoid(q1 + q2), wq_ref[...]) + bq_ref[...]
        a_b = alpha * h_b
        wg_s[...] += _dot_n(oh_g, a_b)
        return carry

    lax.fori_loop(0, NB, phase_c, 0)

    wcat = jnp.concatenate([wl_s[...], wg_s[...]], axis=1)
    w = _dot_t(wcat, wt_ref[...])
    out_ref[...] = _dot_t(w, emb_ref[...])


def _tc_forward(msg01, xi, batchi, embedding, gru_w_ih, gru_w_hh,
                W1, W2, b2r, Wq, bqr, Wt):
    N, H = xi.shape[0], embedding.shape[1]
    NT = embedding.shape[0]
    return pl.pallas_call(
        _tc_body,
        out_shape=jax.ShapeDtypeStruct((G, NT), jnp.float32),
        scratch_shapes=[
            pltpu.VMEM((N, H), jnp.float32),
            pltpu.VMEM((G, H), jnp.float32),
            pltpu.VMEM((G, H), jnp.float32),
            pltpu.VMEM((1, G), jnp.int32),
        ],
    )(msg01, xi, batchi, embedding, gru_w_ih, gru_w_hh, W1, W2, b2r, Wq, bqr, Wt)


def kernel(x, edge_index, batch, embedding, gru_w_ih, gru_w_hh,
           W1, W2, b2, Wq, bq, Wt):
    N = x.shape[0]
    H = embedding.shape[1]
    x_flat = x[:, 0].astype(jnp.int32)
    src = edge_index[0].astype(jnp.int32)
    dst = edge_index[1].astype(jnp.int32)
    msg01 = _sc_edge(x_flat, src, dst, embedding)
    xi = x.astype(jnp.int32).reshape(N, 1)
    batchi = batch.astype(jnp.int32).reshape(N, 1)
    return _tc_forward(msg01, xi, batchi, embedding, gru_w_ih, gru_w_hh,
                       W1, W2, b2.reshape(1, H), Wq, bq.reshape(1, H), Wt)
